# baseline (speedup 1.0000x reference)
import jax
import jax.numpy as jnp
from jax.experimental import pallas as pl
from jax.experimental.pallas import tpu as pltpu


def _body(hour_ref, day_ref, ht_ref, dt_ref, x_ref, o_ref):
    b = pl.program_id(0)
    T = hour_ref.shape[1]
    parts = []
    for t in range(T):
        h = hour_ref[b, t]
        d = day_ref[b, t]
        parts.append(ht_ref[pl.ds(h, 1), :])
        parts.append(dt_ref[pl.ds(d, 1), :])
    emb = jnp.concatenate(parts, axis=1)
    o_ref[0] = x_ref[0] + emb


def kernel(x, hour, day, hour_table, day_table):
    B, N, T, D = x.shape
    TD = T * D
    x3 = x.reshape(B, N, TD)
    hour = hour.astype(jnp.int32)
    day = day.astype(jnp.int32)

    grid_spec = pltpu.PrefetchScalarGridSpec(
        num_scalar_prefetch=2,
        grid=(B,),
        in_specs=[
            pl.BlockSpec(hour_table.shape, lambda b, *_: (0, 0)),
            pl.BlockSpec(day_table.shape, lambda b, *_: (0, 0)),
            pl.BlockSpec((1, N, TD), lambda b, *_: (b, 0, 0)),
        ],
        out_specs=pl.BlockSpec((1, N, TD), lambda b, *_: (b, 0, 0)),
    )
    out = pl.pallas_call(
        _body,
        grid_spec=grid_spec,
        out_shape=jax.ShapeDtypeStruct((B, N, TD), x.dtype),
    )(hour, day, hour_table, day_table, x3)
    return out.reshape(B, N, T, D)


# speedup vs baseline: 1.1402x; 1.1402x over previous
import jax
import jax.numpy as jnp
from jax.experimental import pallas as pl
from jax.experimental.pallas import tpu as pltpu


BB = 4


def _body(hour_ref, day_ref, ht_ref, dt_ref, x_ref, o_ref):
    pid = pl.program_id(0)
    T = hour_ref.shape[1]
    rows = []
    for i in range(BB):
        b = pid * BB + i
        parts = []
        for t in range(T):
            h = hour_ref[b, t]
            d = day_ref[b, t]
            parts.append(ht_ref[pl.ds(h, 1), :])
            parts.append(dt_ref[pl.ds(d, 1), :])
        rows.append(jnp.concatenate(parts, axis=1)[None])
    emb = jnp.concatenate(rows, axis=0)
    o_ref[...] = x_ref[...] + emb


def kernel(x, hour, day, hour_table, day_table):
    B, N, T, D = x.shape
    TD = T * D
    x3 = x.reshape(B, N, TD)
    hour = hour.astype(jnp.int32)
    day = day.astype(jnp.int32)

    grid_spec = pltpu.PrefetchScalarGridSpec(
        num_scalar_prefetch=2,
        grid=(B // BB,),
        in_specs=[
            pl.BlockSpec(hour_table.shape, lambda b, *_: (0, 0)),
            pl.BlockSpec(day_table.shape, lambda b, *_: (0, 0)),
            pl.BlockSpec((BB, N, TD), lambda b, *_: (b, 0, 0)),
        ],
        out_specs=pl.BlockSpec((BB, N, TD), lambda b, *_: (b, 0, 0)),
    )
    out = pl.pallas_call(
        _body,
        grid_spec=grid_spec,
        out_shape=jax.ShapeDtypeStruct((B, N, TD), x.dtype),
    )(hour, day, hour_table, day_table, x3)
    return out.reshape(B, N, T, D)
